# baseline (speedup 1.0000x reference)
import functools

import jax
import jax.numpy as jnp
from jax import lax
from jax.experimental import pallas as pl
from jax.experimental.pallas import tpu as pltpu
from jax.experimental.pallas import tpu_sc as plsc


def _sc_gather(tables, idx_t, n_tokens, n_pad, embed, n_workers, tok_chunk):
    per_w_tok = n_tokens // n_workers
    iters = per_w_tok // tok_chunk
    row_d = n_pad * embed
    n_cat = tables.shape[0]

    mesh = plsc.VectorSubcoreMesh(core_axis_name="c", subcore_axis_name="s")

    @functools.partial(
        pl.kernel,
        out_type=jax.ShapeDtypeStruct((n_tokens, row_d), jnp.float32),
        mesh=mesh,
        scratch_types=[
            pltpu.VMEM((n_pad, tok_chunk), jnp.int32),
            pltpu.VMEM((n_pad * tok_chunk, embed), jnp.float32),
            pltpu.SemaphoreType.DMA,
            pltpu.SemaphoreType.DMA,
        ],
        compiler_params=pltpu.CompilerParams(use_tc_tiling_on_sc=False),
    )
    def k(idx_hbm, tab_hbm, out_hbm, idx_v, rows_v, gsem, ssem):
        n_cores = 2
        wid = lax.axis_index("s") * n_cores + lax.axis_index("c")
        base_chunk = wid * iters

        def body(i, carry):
            chunk_id = base_chunk + i
            tok0 = chunk_id * tok_chunk
            pltpu.sync_copy(idx_hbm.at[pl.ds(chunk_id * n_pad, n_pad)], idx_v)

            def fire_gather(j, c):
                field = jnp.where(j < n_cat, j, j - n_cat)
                pltpu.async_copy(
                    tab_hbm.at[field].at[idx_v.at[j]],
                    rows_v.at[pl.ds(j * tok_chunk, tok_chunk)],
                    gsem,
                )
                return c

            lax.fori_loop(0, n_pad, fire_gather, 0)
            pltpu.make_async_copy(
                tab_hbm.at[0].at[pl.ds(0, n_pad * tok_chunk)], rows_v, gsem
            ).wait()

            def fire_store(j, c):
                pltpu.async_copy(
                    rows_v.at[pl.ds(j * tok_chunk, tok_chunk)],
                    out_hbm.at[pl.ds(tok0, tok_chunk), pl.ds(j * embed, embed)],
                    ssem,
                )
                return c

            lax.fori_loop(0, n_pad, fire_store, 0)
            pltpu.make_async_copy(
                tab_hbm.at[0].at[pl.ds(0, n_pad * tok_chunk)], rows_v, ssem
            ).wait()
            return carry

        lax.fori_loop(0, iters, body, 0)

    return k(idx_t, tables)


def _tc_tail(g, num2d, w_cat, w_num, b_num, w_tail, b_final, block_t):
    t, cat_d = g.shape
    num_dim = num2d.shape[1]
    embed = w_num.shape[1]
    d_model = w_tail.shape[1]
    grid = (t // block_t,)

    def body(g_ref, n_ref, wc_ref, wn_ref, bn_ref, wt_ref, bf_ref, o_ref):
        nump = (
            jnp.dot(n_ref[...], wn_ref[...], preferred_element_type=jnp.float32)
            + bn_ref[...]
        )
        o_ref[...] = (
            jnp.dot(g_ref[...], wc_ref[...], preferred_element_type=jnp.float32)
            + jnp.dot(nump, wt_ref[...], preferred_element_type=jnp.float32)
            + bf_ref[...]
        )

    return pl.pallas_call(
        body,
        grid=grid,
        in_specs=[
            pl.BlockSpec((block_t, cat_d), lambda i: (i, 0)),
            pl.BlockSpec((block_t, num_dim), lambda i: (i, 0)),
            pl.BlockSpec((cat_d, d_model), lambda i: (0, 0)),
            pl.BlockSpec((num_dim, embed), lambda i: (0, 0)),
            pl.BlockSpec((1, embed), lambda i: (0, 0)),
            pl.BlockSpec((embed, d_model), lambda i: (0, 0)),
            pl.BlockSpec((1, d_model), lambda i: (0, 0)),
        ],
        out_specs=pl.BlockSpec((block_t, d_model), lambda i: (i, 0)),
        out_shape=jax.ShapeDtypeStruct((t, d_model), jnp.float32),
    )(
        g,
        num2d,
        w_cat,
        w_num,
        b_num.reshape(1, embed),
        w_tail,
        b_final.reshape(1, d_model),
    )


def kernel(cat_feats, num_feats, tables, W_num, b_num, W_final, b_final):
    b, l, n_cat = cat_feats.shape
    _, vocab, embed = tables.shape
    num_dim = num_feats.shape[-1]
    d_model = W_final.shape[1]
    t = b * l
    n_pad = 28

    gidx = cat_feats.reshape(t, n_cat).astype(jnp.int32)
    gidx = jnp.concatenate([gidx, gidx[:, : n_pad - n_cat]], axis=1)
    tok_chunk = 128
    idx_t = (
        gidx.reshape(t // tok_chunk, tok_chunk, n_pad)
        .transpose(0, 2, 1)
        .reshape(t // tok_chunk * n_pad, tok_chunk)
    )

    g = _sc_gather(
        tables, idx_t, t, n_pad, embed, n_workers=32, tok_chunk=tok_chunk
    )

    cat_d = n_cat * embed
    w_cat = jnp.zeros((n_pad * embed, d_model), jnp.float32).at[:cat_d].set(
        W_final[:cat_d]
    )
    w_tail = W_final[cat_d:]

    out = _tc_tail(
        g, num_feats.reshape(t, num_dim), w_cat, W_num, b_num, w_tail, b_final,
        block_t=2048,
    )
    return out.reshape(b, l, d_model)


# speedup vs baseline: 1.2422x; 1.2422x over previous
import functools

import jax
import jax.numpy as jnp
from jax import lax
from jax.experimental import pallas as pl
from jax.experimental.layout import Format, Layout, with_layout_constraint
from jax.experimental.pallas import tpu as pltpu
from jax.experimental.pallas import tpu_sc as plsc


def _sc_gather(tables, idx_t, n_tokens, n_pad, embed, n_workers, tok_chunk):
    per_w_tok = n_tokens // n_workers
    iters = per_w_tok // tok_chunk
    row_d = n_pad * embed
    n_cat = tables.shape[0]

    mesh = plsc.VectorSubcoreMesh(core_axis_name="c", subcore_axis_name="s")

    @functools.partial(
        pl.kernel,
        out_type=jax.ShapeDtypeStruct((n_tokens, row_d), jnp.float32),
        mesh=mesh,
        scratch_types=[
            pltpu.VMEM((n_pad, tok_chunk), jnp.int32),
            pltpu.VMEM((n_pad * tok_chunk, embed), jnp.float32),
            pltpu.SemaphoreType.DMA,
            pltpu.SemaphoreType.DMA,
        ],
        compiler_params=pltpu.CompilerParams(use_tc_tiling_on_sc=False),
    )
    def k(idx_hbm, tab_hbm, out_hbm, idx_v, rows_v, gsem, ssem):
        n_cores = 2
        wid = lax.axis_index("s") * n_cores + lax.axis_index("c")
        base_chunk = wid * iters

        def body(i, carry):
            chunk_id = base_chunk + i
            tok0 = chunk_id * tok_chunk
            pltpu.sync_copy(idx_hbm.at[pl.ds(chunk_id * n_pad, n_pad)], idx_v)

            def fire_gather(j, c):
                field = jnp.where(j < n_cat, j, j - n_cat)
                pltpu.async_copy(
                    tab_hbm.at[field].at[idx_v.at[j]],
                    rows_v.at[pl.ds(j * tok_chunk, tok_chunk)],
                    gsem,
                )
                return c

            lax.fori_loop(0, n_pad, fire_gather, 0)
            pltpu.make_async_copy(
                tab_hbm.at[0].at[pl.ds(0, n_pad * tok_chunk)], rows_v, gsem
            ).wait()

            def fire_store(j, c):
                pltpu.async_copy(
                    rows_v.at[pl.ds(j * tok_chunk, tok_chunk)],
                    out_hbm.at[pl.ds(tok0, tok_chunk), pl.ds(j * embed, embed)],
                    ssem,
                )
                return c

            lax.fori_loop(0, n_pad, fire_store, 0)
            pltpu.make_async_copy(
                tab_hbm.at[0].at[pl.ds(0, n_pad * tok_chunk)], rows_v, ssem
            ).wait()
            return carry

        lax.fori_loop(0, iters, body, 0)

    return k(idx_t, tables)


def _tc_tail(g, num2d, w_cat, w_num, b_num, w_tail, b_final, block_t):
    t, cat_d = g.shape
    num_dim = num2d.shape[1]
    embed = w_num.shape[1]
    d_model = w_tail.shape[1]
    grid = (t // block_t,)

    def body(g_ref, n_ref, wc_ref, wn_ref, bn_ref, wt_ref, bf_ref, o_ref):
        nump = (
            jnp.dot(n_ref[...], wn_ref[...], preferred_element_type=jnp.float32)
            + bn_ref[...]
        )
        o_ref[...] = (
            jnp.dot(g_ref[...], wc_ref[...], preferred_element_type=jnp.float32)
            + jnp.dot(nump, wt_ref[...], preferred_element_type=jnp.float32)
            + bf_ref[...]
        )

    return pl.pallas_call(
        body,
        grid=grid,
        in_specs=[
            pl.BlockSpec((block_t, cat_d), lambda i: (i, 0)),
            pl.BlockSpec((block_t, num_dim), lambda i: (i, 0)),
            pl.BlockSpec((cat_d, d_model), lambda i: (0, 0)),
            pl.BlockSpec((num_dim, embed), lambda i: (0, 0)),
            pl.BlockSpec((1, embed), lambda i: (0, 0)),
            pl.BlockSpec((embed, d_model), lambda i: (0, 0)),
            pl.BlockSpec((1, d_model), lambda i: (0, 0)),
        ],
        out_specs=pl.BlockSpec((block_t, d_model), lambda i: (i, 0)),
        out_shape=jax.ShapeDtypeStruct((t, d_model), jnp.float32),
    )(
        g,
        num2d,
        w_cat,
        w_num,
        b_num.reshape(1, embed),
        w_tail,
        b_final.reshape(1, d_model),
    )


def kernel(cat_feats, num_feats, tables, W_num, b_num, W_final, b_final):
    b, l, n_cat = cat_feats.shape
    _, vocab, embed = tables.shape
    num_dim = num_feats.shape[-1]
    d_model = W_final.shape[1]
    t = b * l
    n_pad = 28

    gidx = cat_feats.reshape(t, n_cat).astype(jnp.int32)
    gidx = jnp.concatenate([gidx, gidx[:, : n_pad - n_cat]], axis=1)
    tok_chunk = 128
    idx_t = (
        gidx.reshape(t // tok_chunk, tok_chunk, n_pad)
        .transpose(0, 2, 1)
        .reshape(t // tok_chunk * n_pad, tok_chunk)
    )

    tables_lin = with_layout_constraint(
        tables, Layout(major_to_minor=(0, 1, 2), tiling=((8,),))
    )

    g = _sc_gather(
        tables_lin, idx_t, t, n_pad, embed, n_workers=32, tok_chunk=tok_chunk
    )

    cat_d = n_cat * embed
    w_cat = jnp.zeros((n_pad * embed, d_model), jnp.float32).at[:cat_d].set(
        W_final[:cat_d]
    )
    w_tail = W_final[cat_d:]

    out = _tc_tail(
        g, num_feats.reshape(t, num_dim), w_cat, W_num, b_num, w_tail, b_final,
        block_t=2048,
    )
    return out.reshape(b, l, d_model)
